# baseline (speedup 1.0000x reference)
import jax
import jax.numpy as jnp
from jax import lax
from jax.experimental import pallas as pl
from jax.experimental.pallas import tpu as pltpu
from jax.experimental.pallas import tpu_sc as plsc

NEG = -1e9
IOU_T = 0.5
SCORE_T = 0.05
NBOX = 5000
NPAD = 5120
NBLK = NPAD // 16
NCLS = 20
NIMG = 4
KMAX = 100
KPAD = 128
BIG = 2**30


def _softmax_body(conf_ref, box_ref, scores_ref, coords_ref):
    x = jnp.transpose(conf_ref[0], (1, 0))
    m = jnp.max(x, axis=0, keepdims=True)
    e = jnp.exp(x - m)
    p = e / jnp.sum(e, axis=0, keepdims=True)
    col = lax.broadcasted_iota(jnp.int32, (NCLS, NPAD), 1)
    keep = (col < NBOX) & (p > SCORE_T)
    scores_ref[0] = jnp.where(keep, p, NEG)
    coords_ref[0] = jnp.transpose(box_ref[0], (1, 0))


def _spl_f(x):
    return jnp.full((16,), x, jnp.float32)


def _spl_i(x):
    return jnp.full((16,), x, jnp.int32)


def _nms_body(scores_hbm, coords_hbm, outb_hbm, outs_hbm, outc_hbm, outn_hbm,
              sref, cref, bmax, work, ob_v, os_v, oc_v, on_v, mrg, shared):
    cid = lax.axis_index("c")
    sid = lax.axis_index("s")
    lanes = lax.iota(jnp.int32, 16)
    lane0 = lanes == 0

    def run_problem(p):
        img_local = p // NCLS
        cls = p % NCLS
        img = cid * 2 + img_local
        pltpu.sync_copy(scores_hbm.at[img, cls], sref)
        pltpu.sync_copy(coords_hbm.at[img], cref)
        for r in range(6):
            fill = NEG if r == 0 else 0.0
            for t in range(KPAD // 16):
                work[pl.ds(r * KPAD + t * 16, 16)] = jnp.full((16,), fill,
                                                             jnp.float32)

        def build_bm(j, carry):
            base = j * 256 + lanes * 16
            acc = plsc.load_gather(sref, [base])
            for c in range(1, 16):
                acc = jnp.maximum(acc, plsc.load_gather(sref, [base + c]))
            bmax[pl.ds(j * 16, 16)] = acc
            return carry
        lax.fori_loop(0, NBLK // 16, build_bm, 0)

        def cond(st):
            cnt, done = st
            return (cnt < KMAX) & jnp.logical_not(done)

        def body(st):
            cnt, done = st
            m = bmax[pl.ds(0, 16)]
            bidx = jnp.zeros((16,), jnp.int32)
            for j in range(1, NBLK // 16):
                v = bmax[pl.ds(j * 16, 16)]
                upd = v > m
                m = jnp.where(upd, v, m)
                bidx = jnp.where(upd, j, bidx)
            gm = jnp.max(m)
            valid = gm > SCORE_T
            e = jnp.min(jnp.where(m == gm, bidx * 16 + lanes, BIG))
            blk = sref[pl.ds(e * 16, 16)]
            l = jnp.min(jnp.where(blk == gm, lanes, BIG))
            idx = e * 16 + l
            idxv = _spl_i(idx)
            cx1 = plsc.load_gather(cref, [idxv])
            cy1 = plsc.load_gather(cref, [idxv + NPAD])
            cx2 = plsc.load_gather(cref, [idxv + 2 * NPAD])
            cy2 = plsc.load_gather(cref, [idxv + 3 * NPAD])
            carea = jnp.maximum(cx2 - cx1, 0.0) * jnp.maximum(cy2 - cy1, 0.0)

            def supp_blk(t, sv):
                x1 = work[pl.ds(KPAD + t * 16, 16)]
                y1 = work[pl.ds(2 * KPAD + t * 16, 16)]
                x2 = work[pl.ds(3 * KPAD + t * 16, 16)]
                y2 = work[pl.ds(4 * KPAD + t * 16, 16)]
                ar = work[pl.ds(5 * KPAD + t * 16, 16)]
                iw = jnp.maximum(jnp.minimum(cx2, x2) - jnp.maximum(cx1, x1), 0.0)
                ih = jnp.maximum(jnp.minimum(cy2, y2) - jnp.maximum(cy1, y1), 0.0)
                inter = ih * iw
                union = carea + ar - inter
                iou = inter / jnp.maximum(union, 1e-9)
                return sv | (iou > IOU_T)
            nblk_sel = (cnt + 15) >> 4
            sv = lax.fori_loop(0, nblk_sel, supp_blk, jnp.zeros((16,), jnp.bool_))
            suppressed = jnp.any(sv)

            @pl.when(valid)
            def _():
                plsc.store_scatter(sref, [idxv], _spl_f(NEG), mask=lane0)
                nm = jnp.max(jnp.where(lanes == l, NEG, blk))
                plsc.store_scatter(bmax, [_spl_i(e)], _spl_f(nm), mask=lane0)

            take = valid & jnp.logical_not(suppressed)

            @pl.when(take)
            def _():
                cntv = _spl_i(cnt)
                plsc.store_scatter(work, [cntv], _spl_f(gm), mask=lane0)
                plsc.store_scatter(work, [cntv + KPAD], cx1, mask=lane0)
                plsc.store_scatter(work, [cntv + 2 * KPAD], cy1, mask=lane0)
                plsc.store_scatter(work, [cntv + 3 * KPAD], cx2, mask=lane0)
                plsc.store_scatter(work, [cntv + 4 * KPAD], cy2, mask=lane0)
                plsc.store_scatter(work, [cntv + 5 * KPAD], carea, mask=lane0)

            return jnp.where(take, cnt + 1, cnt), jnp.logical_not(valid)

        lax.while_loop(cond, body, (jnp.int32(0), jnp.zeros((), jnp.bool_)))
        pltpu.sync_copy(work, shared.at[pl.ds(p * (6 * KPAD), 6 * KPAD)])

    for k in range(2):
        run_problem(sid + 16 * k)

    @pl.when(sid < 8)
    def _():
        run_problem(sid + 32)

    plsc.subcore_barrier()

    @pl.when(sid >= 14)
    def _():
        iml = sid - 14
        img = cid * 2 + iml
        pltpu.sync_copy(
            shared.at[pl.ds(iml * (NCLS * 6 * KPAD), NCLS * 6 * KPAD)], mrg)
        zeros16 = jnp.zeros((16,), jnp.int32)
        neg1 = jnp.full((16,), -1.0, jnp.float32)
        for t in range(KPAD // 16):
            os_v[pl.ds(t * 16, 16)] = neg1
            oc_v[pl.ds(t * 16, 16)] = neg1
        for t in range(KPAD * 4 // 16):
            ob_v[pl.ds(t * 16, 16)] = neg1
        cls_hi_ok = lanes < (NCLS - 16)
        cls1 = jnp.where(cls_hi_ok, lanes + 16, 0)

        def mcond(st):
            t, done, _, _ = st
            return (t < KMAX) & jnp.logical_not(done)

        def mbody(st):
            t, done, h0, h1 = st
            slot1 = jnp.where(cls_hi_ok, h1, KPAD - 1)
            v0 = plsc.load_gather(mrg, [lanes * (6 * KPAD) + h0])
            v1 = plsc.load_gather(mrg, [cls1 * (6 * KPAD) + slot1])
            gm = jnp.maximum(jnp.max(v0), jnp.max(v1))
            valid = gm > SCORE_T
            k0 = jnp.min(jnp.where(v0 == gm, lanes * KPAD + h0, BIG))
            k1 = jnp.min(jnp.where(v1 == gm, cls1 * KPAD + slot1, BIG))
            kbest = jnp.minimum(k0, k1)
            cbest = kbest // KPAD
            sbest = kbest % KPAD

            @pl.when(valid)
            def _():
                tv = _spl_i(t)
                plsc.store_scatter(os_v, [tv], _spl_f(gm), mask=lane0)
                plsc.store_scatter(oc_v, [tv],
                                   _spl_f(cbest.astype(jnp.float32)), mask=lane0)
                rows = 1 + (lanes & 3)
                vals = plsc.load_gather(
                    mrg, [cbest * (6 * KPAD) + rows * KPAD + sbest])
                plsc.store_scatter(ob_v, [t * 4 + (lanes & 3)], vals,
                                   mask=lanes < 4)

            upd = jnp.where(valid, jnp.int32(1), jnp.int32(0))
            h0n = h0 + jnp.where(lanes == cbest, upd, 0)
            h1n = h1 + jnp.where(lanes + 16 == cbest, upd, 0)
            return (jnp.where(valid, t + 1, t), jnp.logical_not(valid), h0n, h1n)

        tf, _, _, _ = lax.while_loop(
            mcond, mbody, (jnp.int32(0), jnp.zeros((), jnp.bool_), zeros16, zeros16))
        on_v[...] = jnp.where(lane0, tf, 0)
        pltpu.sync_copy(ob_v, outb_hbm.at[img])
        pltpu.sync_copy(os_v, outs_hbm.at[img])
        pltpu.sync_copy(oc_v, outc_hbm.at[img])
        pltpu.sync_copy(on_v, outn_hbm.at[img])


def kernel(box_prediction, confidence_prediction):
    conf_p = jnp.pad(confidence_prediction, ((0, 0), (0, NPAD - NBOX), (0, 0)))
    box_p = jnp.pad(box_prediction, ((0, 0), (0, NPAD - NBOX), (0, 0)))

    scores, coords3 = pl.pallas_call(
        _softmax_body,
        grid=(NIMG,),
        in_specs=[
            pl.BlockSpec((1, NPAD, NCLS), lambda i: (i, 0, 0)),
            pl.BlockSpec((1, NPAD, 4), lambda i: (i, 0, 0)),
        ],
        out_specs=[
            pl.BlockSpec((1, NCLS, NPAD), lambda i: (i, 0, 0)),
            pl.BlockSpec((1, 4, NPAD), lambda i: (i, 0, 0)),
        ],
        out_shape=[
            jax.ShapeDtypeStruct((NIMG, NCLS, NPAD), jnp.float32),
            jax.ShapeDtypeStruct((NIMG, 4, NPAD), jnp.float32),
        ],
    )(conf_p, box_p)
    coords = coords3.reshape(NIMG, 4 * NPAD)

    mesh = plsc.VectorSubcoreMesh(core_axis_name="c", subcore_axis_name="s")
    nms = pl.kernel(
        _nms_body,
        mesh=mesh,
        compiler_params=pltpu.CompilerParams(needs_layout_passes=False),
        out_type=[
            jax.ShapeDtypeStruct((NIMG, KPAD * 4), jnp.float32),
            jax.ShapeDtypeStruct((NIMG, KPAD), jnp.float32),
            jax.ShapeDtypeStruct((NIMG, KPAD), jnp.float32),
            jax.ShapeDtypeStruct((NIMG, 16), jnp.int32),
        ],
        scratch_types=[
            pltpu.VMEM((NPAD,), jnp.float32),
            pltpu.VMEM((4 * NPAD,), jnp.float32),
            pltpu.VMEM((NBLK,), jnp.float32),
            pltpu.VMEM((6 * KPAD,), jnp.float32),
            pltpu.VMEM((KPAD * 4,), jnp.float32),
            pltpu.VMEM((KPAD,), jnp.float32),
            pltpu.VMEM((KPAD,), jnp.float32),
            pltpu.VMEM((16,), jnp.int32),
            pltpu.VMEM((NCLS * 6 * KPAD,), jnp.float32),
            pltpu.VMEM_SHARED((40 * 6 * KPAD,), jnp.float32),
        ],
    )
    outb, outs, outc, outn = nms(scores, coords)
    out_b = outb.reshape(NIMG, KPAD, 4)[:, :KMAX, :]
    out_s = outs[:, :KMAX]
    out_c = outc[:, :KMAX]
    num_det = outn[:, 0]
    return out_b, out_s, out_c, num_det


# speedup vs baseline: 1.3960x; 1.3960x over previous
import jax
import jax.numpy as jnp
from jax import lax
from jax.experimental import pallas as pl
from jax.experimental.pallas import tpu as pltpu
from jax.experimental.pallas import tpu_sc as plsc

NEG = -1e9
IOU_T = 0.5
SCORE_T = 0.05
NBOX = 5000
NPAD = 5120
NBLK = NPAD // 16
NCLS = 20
NIMG = 4
KMAX = 100
KPAD = 128
BIG = 2**30


def _softmax_body(conf_ref, scores_ref):
    x = conf_ref[0]
    m = jnp.max(x, axis=0, keepdims=True)
    e = jnp.exp(x - m)
    p = e / jnp.sum(e, axis=0, keepdims=True)
    col = lax.broadcasted_iota(jnp.int32, (NCLS, NPAD), 1)
    keep = (col < NBOX) & (p > SCORE_T)
    scores_ref[0] = jnp.where(keep, p, NEG)


def _spl_f(x):
    return jnp.full((16,), x, jnp.float32)


def _spl_i(x):
    return jnp.full((16,), x, jnp.int32)


def _nms_body(scores_hbm, coords_hbm, outb_hbm, outs_hbm, outc_hbm, outn_hbm,
              sref0, cref0, sref1, cref1, sem0, sem1,
              bmax, work, ob_v, os_v, oc_v, on_v, mrg, shared):
    cid = lax.axis_index("c")
    sid = lax.axis_index("s")
    lanes = lax.iota(jnp.int32, 16)
    lane0 = lanes == 0

    def hbm_refs(p):
        img_local = p // NCLS
        cls = p % NCLS
        img = cid * 2 + img_local
        return scores_hbm.at[img, cls], coords_hbm.at[img], img

    def start_load(p, sref, cref, sem):
        s_hbm, c_hbm, _ = hbm_refs(p)
        pltpu.async_copy(s_hbm, sref, sem)
        pltpu.async_copy(c_hbm, cref, sem)

    def wait_load(p, sref, cref, sem):
        s_hbm, c_hbm, _ = hbm_refs(p)
        pltpu.make_async_copy(s_hbm, sref, sem).wait()
        pltpu.make_async_copy(c_hbm, cref, sem).wait()

    def run_problem(p, sref, cref):
        for r in range(6):
            fill = NEG if r == 0 else 0.0
            for t in range(KPAD // 16):
                work[pl.ds(r * KPAD + t * 16, 16)] = jnp.full((16,), fill,
                                                             jnp.float32)

        def build_bm(j, carry):
            base = j * 256 + lanes * 16
            acc = plsc.load_gather(sref, [base])
            for c in range(1, 16):
                acc = jnp.maximum(acc, plsc.load_gather(sref, [base + c]))
            bmax[pl.ds(j * 16, 16)] = acc
            return carry
        lax.fori_loop(0, NBLK // 16, build_bm, 0)

        def cond(st):
            cnt, done = st
            return (cnt < KMAX) & jnp.logical_not(done)

        def body(st):
            cnt, done = st
            m = bmax[pl.ds(0, 16)]
            bidx = jnp.zeros((16,), jnp.int32)
            for j in range(1, NBLK // 16):
                v = bmax[pl.ds(j * 16, 16)]
                upd = v > m
                m = jnp.where(upd, v, m)
                bidx = jnp.where(upd, j, bidx)
            gm = jnp.max(m)
            valid = gm > SCORE_T
            e = jnp.min(jnp.where(m == gm, bidx * 16 + lanes, BIG))
            blk = sref[pl.ds(e * 16, 16)]
            l = jnp.min(jnp.where(blk == gm, lanes, BIG))
            idx = e * 16 + l
            idxv = _spl_i(idx)
            cx1 = plsc.load_gather(cref, [idxv])
            cy1 = plsc.load_gather(cref, [idxv + NPAD])
            cx2 = plsc.load_gather(cref, [idxv + 2 * NPAD])
            cy2 = plsc.load_gather(cref, [idxv + 3 * NPAD])
            carea = jnp.maximum(cx2 - cx1, 0.0) * jnp.maximum(cy2 - cy1, 0.0)

            def supp_blk(t, sv):
                x1 = work[pl.ds(KPAD + t * 16, 16)]
                y1 = work[pl.ds(2 * KPAD + t * 16, 16)]
                x2 = work[pl.ds(3 * KPAD + t * 16, 16)]
                y2 = work[pl.ds(4 * KPAD + t * 16, 16)]
                ar = work[pl.ds(5 * KPAD + t * 16, 16)]
                iw = jnp.maximum(jnp.minimum(cx2, x2) - jnp.maximum(cx1, x1), 0.0)
                ih = jnp.maximum(jnp.minimum(cy2, y2) - jnp.maximum(cy1, y1), 0.0)
                inter = ih * iw
                union = carea + ar - inter
                iou = inter / jnp.maximum(union, 1e-9)
                return sv | (iou > IOU_T)
            nblk_sel = (cnt + 15) >> 4
            sv = lax.fori_loop(0, nblk_sel, supp_blk, jnp.zeros((16,), jnp.bool_))
            suppressed = jnp.any(sv)

            @pl.when(valid)
            def _():
                plsc.store_scatter(sref, [idxv], _spl_f(NEG), mask=lane0)
                nm = jnp.max(jnp.where(lanes == l, NEG, blk))
                plsc.store_scatter(bmax, [_spl_i(e)], _spl_f(nm), mask=lane0)

            take = valid & jnp.logical_not(suppressed)

            @pl.when(take)
            def _():
                cntv = _spl_i(cnt)
                plsc.store_scatter(work, [cntv], _spl_f(gm), mask=lane0)
                plsc.store_scatter(work, [cntv + KPAD], cx1, mask=lane0)
                plsc.store_scatter(work, [cntv + 2 * KPAD], cy1, mask=lane0)
                plsc.store_scatter(work, [cntv + 3 * KPAD], cx2, mask=lane0)
                plsc.store_scatter(work, [cntv + 4 * KPAD], cy2, mask=lane0)
                plsc.store_scatter(work, [cntv + 5 * KPAD], carea, mask=lane0)

            return jnp.where(take, cnt + 1, cnt), jnp.logical_not(valid)

        lax.while_loop(cond, body, (jnp.int32(0), jnp.zeros((), jnp.bool_)))
        pltpu.sync_copy(work, shared.at[pl.ds(p * (6 * KPAD), 6 * KPAD)])

    p0, p1, p2 = sid, sid + 16, sid + 32
    start_load(p0, sref0, cref0, sem0)
    start_load(p1, sref1, cref1, sem1)
    wait_load(p0, sref0, cref0, sem0)
    run_problem(p0, sref0, cref0)

    @pl.when(sid < 8)
    def _():
        start_load(p2, sref0, cref0, sem0)

    wait_load(p1, sref1, cref1, sem1)
    run_problem(p1, sref1, cref1)

    @pl.when(sid < 8)
    def _():
        wait_load(p2, sref0, cref0, sem0)
        run_problem(p2, sref0, cref0)

    plsc.subcore_barrier()

    @pl.when(sid >= 14)
    def _():
        iml = sid - 14
        img = cid * 2 + iml
        pltpu.sync_copy(
            shared.at[pl.ds(iml * (NCLS * 6 * KPAD), NCLS * 6 * KPAD)], mrg)
        zeros16 = jnp.zeros((16,), jnp.int32)
        neg1 = jnp.full((16,), -1.0, jnp.float32)
        for t in range(KPAD // 16):
            os_v[pl.ds(t * 16, 16)] = neg1
            oc_v[pl.ds(t * 16, 16)] = neg1
        for t in range(KPAD * 4 // 16):
            ob_v[pl.ds(t * 16, 16)] = neg1
        cls_hi_ok = lanes < (NCLS - 16)
        cls1 = jnp.where(cls_hi_ok, lanes + 16, 0)

        def mcond(st):
            t, done, _, _ = st
            return (t < KMAX) & jnp.logical_not(done)

        def mbody(st):
            t, done, h0, h1 = st
            slot1 = jnp.where(cls_hi_ok, h1, KPAD - 1)
            v0 = plsc.load_gather(mrg, [lanes * (6 * KPAD) + h0])
            v1 = plsc.load_gather(mrg, [cls1 * (6 * KPAD) + slot1])
            gm = jnp.maximum(jnp.max(v0), jnp.max(v1))
            valid = gm > SCORE_T
            k0 = jnp.min(jnp.where(v0 == gm, lanes * KPAD + h0, BIG))
            k1 = jnp.min(jnp.where(v1 == gm, cls1 * KPAD + slot1, BIG))
            kbest = jnp.minimum(k0, k1)
            cbest = kbest // KPAD
            sbest = kbest % KPAD

            @pl.when(valid)
            def _():
                tv = _spl_i(t)
                plsc.store_scatter(os_v, [tv], _spl_f(gm), mask=lane0)
                plsc.store_scatter(oc_v, [tv],
                                   _spl_f(cbest.astype(jnp.float32)), mask=lane0)
                rows = 1 + (lanes & 3)
                vals = plsc.load_gather(
                    mrg, [cbest * (6 * KPAD) + rows * KPAD + sbest])
                plsc.store_scatter(ob_v, [t * 4 + (lanes & 3)], vals,
                                   mask=lanes < 4)

            upd = jnp.where(valid, jnp.int32(1), jnp.int32(0))
            h0n = h0 + jnp.where(lanes == cbest, upd, 0)
            h1n = h1 + jnp.where(lanes + 16 == cbest, upd, 0)
            return (jnp.where(valid, t + 1, t), jnp.logical_not(valid), h0n, h1n)

        tf, _, _, _ = lax.while_loop(
            mcond, mbody, (jnp.int32(0), jnp.zeros((), jnp.bool_), zeros16, zeros16))
        on_v[...] = jnp.where(lane0, tf, 0)
        pltpu.sync_copy(ob_v, outb_hbm.at[img])
        pltpu.sync_copy(os_v, outs_hbm.at[img])
        pltpu.sync_copy(oc_v, outc_hbm.at[img])
        pltpu.sync_copy(on_v, outn_hbm.at[img])


def kernel(box_prediction, confidence_prediction):
    conf_t = jnp.transpose(confidence_prediction, (0, 2, 1))
    conf_t = jnp.pad(conf_t, ((0, 0), (0, 0), (0, NPAD - NBOX)))
    coords = jnp.transpose(box_prediction, (0, 2, 1))
    coords = jnp.pad(coords, ((0, 0), (0, 0), (0, NPAD - NBOX)))
    coords = coords.reshape(NIMG, 4 * NPAD)

    scores = pl.pallas_call(
        _softmax_body,
        grid=(NIMG,),
        in_specs=[pl.BlockSpec((1, NCLS, NPAD), lambda i: (i, 0, 0))],
        out_specs=pl.BlockSpec((1, NCLS, NPAD), lambda i: (i, 0, 0)),
        out_shape=jax.ShapeDtypeStruct((NIMG, NCLS, NPAD), jnp.float32),
    )(conf_t)

    mesh = plsc.VectorSubcoreMesh(core_axis_name="c", subcore_axis_name="s")
    nms = pl.kernel(
        _nms_body,
        mesh=mesh,
        compiler_params=pltpu.CompilerParams(needs_layout_passes=False),
        out_type=[
            jax.ShapeDtypeStruct((NIMG, KPAD * 4), jnp.float32),
            jax.ShapeDtypeStruct((NIMG, KPAD), jnp.float32),
            jax.ShapeDtypeStruct((NIMG, KPAD), jnp.float32),
            jax.ShapeDtypeStruct((NIMG, 16), jnp.int32),
        ],
        scratch_types=[
            pltpu.VMEM((NPAD,), jnp.float32),
            pltpu.VMEM((4 * NPAD,), jnp.float32),
            pltpu.VMEM((NPAD,), jnp.float32),
            pltpu.VMEM((4 * NPAD,), jnp.float32),
            pltpu.SemaphoreType.DMA,
            pltpu.SemaphoreType.DMA,
            pltpu.VMEM((NBLK,), jnp.float32),
            pltpu.VMEM((6 * KPAD,), jnp.float32),
            pltpu.VMEM((KPAD * 4,), jnp.float32),
            pltpu.VMEM((KPAD,), jnp.float32),
            pltpu.VMEM((KPAD,), jnp.float32),
            pltpu.VMEM((16,), jnp.int32),
            pltpu.VMEM((NCLS * 6 * KPAD,), jnp.float32),
            pltpu.VMEM_SHARED((40 * 6 * KPAD,), jnp.float32),
        ],
    )
    outb, outs, outc, outn = nms(scores, coords)
    out_b = outb.reshape(NIMG, KPAD, 4)[:, :KMAX, :]
    out_s = outs[:, :KMAX]
    out_c = outc[:, :KMAX]
    num_det = outn[:, 0]
    return out_b, out_s, out_c, num_det


# speedup vs baseline: 1.4982x; 1.0732x over previous
import jax
import jax.numpy as jnp
from jax import lax
from jax.experimental import pallas as pl
from jax.experimental.pallas import tpu as pltpu
from jax.experimental.pallas import tpu_sc as plsc

NEG = -1e9
IOU_T = 0.5
SCORE_T = 0.05
NBOX = 5000
NPAD = 5120
NBLK = NPAD // 16
NCLS = 20
NIMG = 4
KMAX = 100
KPAD = 128
BIG = 2**30


def _softmax_body(conf_ref, scores_ref):
    x = conf_ref[0]
    m = jnp.max(x, axis=0, keepdims=True)
    e = jnp.exp(x - m)
    p = e / jnp.sum(e, axis=0, keepdims=True)
    col = lax.broadcasted_iota(jnp.int32, (NCLS, NPAD), 1)
    keep = (col < NBOX) & (p > SCORE_T)
    scores_ref[0] = jnp.where(keep, p, NEG)


def _spl_f(x):
    return jnp.full((16,), x, jnp.float32)


def _spl_i(x):
    return jnp.full((16,), x, jnp.int32)


def _nms_body(scores_hbm, coords_hbm, outb_hbm, outs_hbm, outc_hbm, outn_hbm,
              sref0, cref0, sref1, cref1, sem0, sem1,
              bmax, work, ob_v, os_v, oc_v, on_v, mrg, shared):
    cid = lax.axis_index("c")
    sid = lax.axis_index("s")
    lanes = lax.iota(jnp.int32, 16)
    lane0 = lanes == 0

    def hbm_refs(p):
        img_local = p // NCLS
        cls = p % NCLS
        img = cid * 2 + img_local
        return scores_hbm.at[img, cls], coords_hbm.at[img], img

    def start_load(p, sref, cref, sem):
        s_hbm, c_hbm, _ = hbm_refs(p)
        pltpu.async_copy(s_hbm, sref, sem)
        pltpu.async_copy(c_hbm, cref, sem)

    def wait_load(p, sref, cref, sem):
        s_hbm, c_hbm, _ = hbm_refs(p)
        pltpu.make_async_copy(s_hbm, sref, sem).wait()
        pltpu.make_async_copy(c_hbm, cref, sem).wait()

    def run_problem(p, sref, cref):
        for r in range(6):
            fill = NEG if r == 0 else 0.0
            for t in range(KPAD // 16):
                work[pl.ds(r * KPAD + t * 16, 16)] = jnp.full((16,), fill,
                                                             jnp.float32)

        def build_bm(j, carry):
            base = j * 256 + lanes * 16
            acc = plsc.load_gather(sref, [base])
            for c in range(1, 16):
                acc = jnp.maximum(acc, plsc.load_gather(sref, [base + c]))
            bmax[pl.ds(j * 16, 16)] = acc
            return carry
        lax.fori_loop(0, NBLK // 16, build_bm, 0)

        def cond(st):
            cnt, done = st
            return (cnt < KMAX) & jnp.logical_not(done)

        def body(st):
            cnt, done = st
            NGRP = NBLK // 16
            chains = []
            for c0 in range(0, NGRP, 5):
                mc = bmax[pl.ds(c0 * 16, 16)]
                bc = jnp.full((16,), c0, jnp.int32)
                for j in range(c0 + 1, min(c0 + 5, NGRP)):
                    v = bmax[pl.ds(j * 16, 16)]
                    upd = v > mc
                    mc = jnp.where(upd, v, mc)
                    bc = jnp.where(upd, j, bc)
                chains.append((mc, bc))
            while len(chains) > 1:
                (ma, ba), (mb, bb) = chains[0], chains[1]
                upd = mb > ma
                chains = chains[2:] + [(jnp.where(upd, mb, ma),
                                        jnp.where(upd, bb, ba))]
            m, bidx = chains[0]
            gm = jnp.max(m)
            valid = gm > SCORE_T
            e = jnp.min(jnp.where(m == gm, bidx * 16 + lanes, BIG))
            blk = sref[pl.ds(e * 16, 16)]
            lv = plsc.all_reduce_ffs(blk == gm)
            idxv16 = e * 16 + lv
            idxv = idxv16
            cx1 = plsc.load_gather(cref, [idxv])
            cy1 = plsc.load_gather(cref, [idxv + NBOX])
            cx2 = plsc.load_gather(cref, [idxv + 2 * NBOX])
            cy2 = plsc.load_gather(cref, [idxv + 3 * NBOX])
            carea = jnp.maximum(cx2 - cx1, 0.0) * jnp.maximum(cy2 - cy1, 0.0)

            def supp_blk(t, sv):
                x1 = work[pl.ds(KPAD + t * 16, 16)]
                y1 = work[pl.ds(2 * KPAD + t * 16, 16)]
                x2 = work[pl.ds(3 * KPAD + t * 16, 16)]
                y2 = work[pl.ds(4 * KPAD + t * 16, 16)]
                ar = work[pl.ds(5 * KPAD + t * 16, 16)]
                iw = jnp.maximum(jnp.minimum(cx2, x2) - jnp.maximum(cx1, x1), 0.0)
                ih = jnp.maximum(jnp.minimum(cy2, y2) - jnp.maximum(cy1, y1), 0.0)
                inter = ih * iw
                union = carea + ar - inter
                iou = inter / jnp.maximum(union, 1e-9)
                return sv | (iou > IOU_T)
            nblk_sel = (cnt + 15) >> 4
            sv = lax.fori_loop(0, nblk_sel, supp_blk, jnp.zeros((16,), jnp.bool_))
            suppressed = jnp.any(sv)

            kmask = lane0 & valid
            plsc.store_scatter(sref, [idxv], _spl_f(NEG), mask=kmask)
            nm = jnp.max(jnp.where(lanes == lv, NEG, blk))
            plsc.store_scatter(bmax, [_spl_i(e)], _spl_f(nm), mask=kmask)

            take = valid & jnp.logical_not(suppressed)
            tmask = lane0 & take
            cntv = _spl_i(cnt)
            plsc.store_scatter(work, [cntv], _spl_f(gm), mask=tmask)
            plsc.store_scatter(work, [cntv + KPAD], cx1, mask=tmask)
            plsc.store_scatter(work, [cntv + 2 * KPAD], cy1, mask=tmask)
            plsc.store_scatter(work, [cntv + 3 * KPAD], cx2, mask=tmask)
            plsc.store_scatter(work, [cntv + 4 * KPAD], cy2, mask=tmask)
            plsc.store_scatter(work, [cntv + 5 * KPAD], carea, mask=tmask)

            return jnp.where(take, cnt + 1, cnt), jnp.logical_not(valid)

        lax.while_loop(cond, body, (jnp.int32(0), jnp.zeros((), jnp.bool_)))
        pltpu.sync_copy(work, shared.at[pl.ds(p * (6 * KPAD), 6 * KPAD)])

    p0, p1, p2 = sid, sid + 16, sid + 32
    start_load(p0, sref0, cref0, sem0)
    start_load(p1, sref1, cref1, sem1)
    wait_load(p0, sref0, cref0, sem0)
    run_problem(p0, sref0, cref0)

    @pl.when(sid < 8)
    def _():
        start_load(p2, sref0, cref0, sem0)

    wait_load(p1, sref1, cref1, sem1)
    run_problem(p1, sref1, cref1)

    @pl.when(sid < 8)
    def _():
        wait_load(p2, sref0, cref0, sem0)
        run_problem(p2, sref0, cref0)

    plsc.subcore_barrier()

    @pl.when(sid >= 14)
    def _():
        iml = sid - 14
        img = cid * 2 + iml
        pltpu.sync_copy(
            shared.at[pl.ds(iml * (NCLS * 6 * KPAD), NCLS * 6 * KPAD)], mrg)
        zeros16 = jnp.zeros((16,), jnp.int32)
        neg1 = jnp.full((16,), -1.0, jnp.float32)
        for t in range(KPAD // 16):
            os_v[pl.ds(t * 16, 16)] = neg1
            oc_v[pl.ds(t * 16, 16)] = neg1
        for t in range(KPAD * 4 // 16):
            ob_v[pl.ds(t * 16, 16)] = neg1
        cls_hi_ok = lanes < (NCLS - 16)
        cls1 = jnp.where(cls_hi_ok, lanes + 16, 0)

        def mcond(st):
            t, done, _, _ = st
            return (t < KMAX) & jnp.logical_not(done)

        def mbody(st):
            t, done, h0, h1 = st
            slot1 = jnp.where(cls_hi_ok, h1, KPAD - 1)
            v0 = plsc.load_gather(mrg, [lanes * (6 * KPAD) + h0])
            v1 = plsc.load_gather(mrg, [cls1 * (6 * KPAD) + slot1])
            gm = jnp.maximum(jnp.max(v0), jnp.max(v1))
            valid = gm > SCORE_T
            k0 = jnp.min(jnp.where(v0 == gm, lanes * KPAD + h0, BIG))
            k1 = jnp.min(jnp.where(v1 == gm, cls1 * KPAD + slot1, BIG))
            kbest = jnp.minimum(k0, k1)
            cbest = kbest // KPAD
            sbest = kbest % KPAD

            @pl.when(valid)
            def _():
                tv = _spl_i(t)
                plsc.store_scatter(os_v, [tv], _spl_f(gm), mask=lane0)
                plsc.store_scatter(oc_v, [tv],
                                   _spl_f(cbest.astype(jnp.float32)), mask=lane0)
                rows = 1 + (lanes & 3)
                vals = plsc.load_gather(
                    mrg, [cbest * (6 * KPAD) + rows * KPAD + sbest])
                plsc.store_scatter(ob_v, [t * 4 + (lanes & 3)], vals,
                                   mask=lanes < 4)

            upd = jnp.where(valid, jnp.int32(1), jnp.int32(0))
            h0n = h0 + jnp.where(lanes == cbest, upd, 0)
            h1n = h1 + jnp.where(lanes + 16 == cbest, upd, 0)
            return (jnp.where(valid, t + 1, t), jnp.logical_not(valid), h0n, h1n)

        tf, _, _, _ = lax.while_loop(
            mcond, mbody, (jnp.int32(0), jnp.zeros((), jnp.bool_), zeros16, zeros16))
        on_v[...] = jnp.where(lane0, tf, 0)
        pltpu.sync_copy(ob_v, outb_hbm.at[img])
        pltpu.sync_copy(os_v, outs_hbm.at[img])
        pltpu.sync_copy(oc_v, outc_hbm.at[img])
        pltpu.sync_copy(on_v, outn_hbm.at[img])


def kernel(box_prediction, confidence_prediction):
    conf_t = jnp.transpose(confidence_prediction, (0, 2, 1))
    conf_t = jnp.pad(conf_t, ((0, 0), (0, 0), (0, NPAD - NBOX)))
    coords = jnp.transpose(box_prediction, (0, 2, 1)).reshape(NIMG, 4 * NBOX)

    scores = pl.pallas_call(
        _softmax_body,
        grid=(NIMG,),
        in_specs=[pl.BlockSpec((1, NCLS, NPAD), lambda i: (i, 0, 0))],
        out_specs=pl.BlockSpec((1, NCLS, NPAD), lambda i: (i, 0, 0)),
        out_shape=jax.ShapeDtypeStruct((NIMG, NCLS, NPAD), jnp.float32),
    )(conf_t)

    mesh = plsc.VectorSubcoreMesh(core_axis_name="c", subcore_axis_name="s")
    nms = pl.kernel(
        _nms_body,
        mesh=mesh,
        compiler_params=pltpu.CompilerParams(needs_layout_passes=False),
        out_type=[
            jax.ShapeDtypeStruct((NIMG, KPAD * 4), jnp.float32),
            jax.ShapeDtypeStruct((NIMG, KPAD), jnp.float32),
            jax.ShapeDtypeStruct((NIMG, KPAD), jnp.float32),
            jax.ShapeDtypeStruct((NIMG, 16), jnp.int32),
        ],
        scratch_types=[
            pltpu.VMEM((NPAD,), jnp.float32),
            pltpu.VMEM((4 * NBOX,), jnp.float32),
            pltpu.VMEM((NPAD,), jnp.float32),
            pltpu.VMEM((4 * NBOX,), jnp.float32),
            pltpu.SemaphoreType.DMA,
            pltpu.SemaphoreType.DMA,
            pltpu.VMEM((NBLK,), jnp.float32),
            pltpu.VMEM((6 * KPAD,), jnp.float32),
            pltpu.VMEM((KPAD * 4,), jnp.float32),
            pltpu.VMEM((KPAD,), jnp.float32),
            pltpu.VMEM((KPAD,), jnp.float32),
            pltpu.VMEM((16,), jnp.int32),
            pltpu.VMEM((NCLS * 6 * KPAD,), jnp.float32),
            pltpu.VMEM_SHARED((40 * 6 * KPAD,), jnp.float32),
        ],
    )
    outb, outs, outc, outn = nms(scores, coords)
    out_b = outb.reshape(NIMG, KPAD, 4)[:, :KMAX, :]
    out_s = outs[:, :KMAX]
    out_c = outc[:, :KMAX]
    num_det = outn[:, 0]
    return out_b, out_s, out_c, num_det


# speedup vs baseline: 1.5700x; 1.0479x over previous
import jax
import jax.numpy as jnp
from jax import lax
from jax.experimental import pallas as pl
from jax.experimental.pallas import tpu as pltpu
from jax.experimental.pallas import tpu_sc as plsc

NEG = -1e9
IOU_T = 0.5
SCORE_T = 0.05
NBOX = 5000
NPAD = 5120
NBLK = NPAD // 16
NCLS = 20
NIMG = 4
KMAX = 100
KPAD = 128
BIG = 2**30


def _softmax_body(conf_ref, scores_ref):
    x = conf_ref[0]
    m = jnp.max(x, axis=0, keepdims=True)
    e = jnp.exp(x - m)
    p = e / jnp.sum(e, axis=0, keepdims=True)
    col = lax.broadcasted_iota(jnp.int32, (NCLS, NPAD), 1)
    keep = (col < NBOX) & (p > SCORE_T)
    scores_ref[0] = jnp.where(keep, p, NEG)


def _spl_f(x):
    return jnp.full((16,), x, jnp.float32)


def _spl_i(x):
    return jnp.full((16,), x, jnp.int32)


def _nms_body(scores_hbm, coords_hbm, outb_hbm, outs_hbm, outc_hbm, outn_hbm,
              sref0, cref0, sref1, cref1, sem0, sem1,
              bmax, work, ob_v, os_v, oc_v, on_v, mrg, shared):
    cid = lax.axis_index("c")
    sid = lax.axis_index("s")
    lanes = lax.iota(jnp.int32, 16)
    lane0 = lanes == 0

    def hbm_refs(p):
        img_local = p // NCLS
        cls = p % NCLS
        img = cid * 2 + img_local
        return scores_hbm.at[img, cls], coords_hbm.at[img], img

    def start_load(p, sref, cref, sem):
        s_hbm, c_hbm, _ = hbm_refs(p)
        pltpu.async_copy(s_hbm, sref, sem)
        pltpu.async_copy(c_hbm, cref, sem)

    def wait_load(p, sref, cref, sem):
        s_hbm, c_hbm, _ = hbm_refs(p)
        pltpu.make_async_copy(s_hbm, sref, sem).wait()
        pltpu.make_async_copy(c_hbm, cref, sem).wait()

    def run_problem(p, sref, cref):
        for r in range(6):
            fill = NEG if r == 0 else 0.0
            for t in range(KPAD // 16):
                work[pl.ds(r * KPAD + t * 16, 16)] = jnp.full((16,), fill,
                                                             jnp.float32)

        def build_bm(j, carry):
            base = j * 256 + lanes * 16
            acc = plsc.load_gather(sref, [base])
            for c in range(1, 16):
                acc = jnp.maximum(acc, plsc.load_gather(sref, [base + c]))
            bmax[pl.ds(j * 16, 16)] = acc
            return carry
        lax.fori_loop(0, NBLK // 16, build_bm, 0)

        def cond(st):
            cnt, done = st
            return (cnt < KMAX) & jnp.logical_not(done)

        def body(st):
            cnt, done = st
            NGRP = NBLK // 16
            chains = []
            for c0 in range(0, NGRP, 5):
                mc = bmax[pl.ds(c0 * 16, 16)]
                bc = jnp.full((16,), c0, jnp.int32)
                for j in range(c0 + 1, min(c0 + 5, NGRP)):
                    v = bmax[pl.ds(j * 16, 16)]
                    upd = v > mc
                    mc = jnp.where(upd, v, mc)
                    bc = jnp.where(upd, j, bc)
                chains.append((mc, bc))
            while len(chains) > 1:
                (ma, ba), (mb, bb) = chains[0], chains[1]
                upd = mb > ma
                chains = chains[2:] + [(jnp.where(upd, mb, ma),
                                        jnp.where(upd, bb, ba))]
            m, bidx = chains[0]
            gm = jnp.max(m)
            valid = gm > SCORE_T
            e = jnp.min(jnp.where(m == gm, bidx * 16 + lanes, BIG))
            blk = sref[pl.ds(e * 16, 16)]
            lv = plsc.all_reduce_ffs(blk == gm)
            idxv16 = e * 16 + lv
            idxv = idxv16
            cx1 = plsc.load_gather(cref, [idxv])
            cy1 = plsc.load_gather(cref, [idxv + NBOX])
            cx2 = plsc.load_gather(cref, [idxv + 2 * NBOX])
            cy2 = plsc.load_gather(cref, [idxv + 3 * NBOX])
            carea = jnp.maximum(cx2 - cx1, 0.0) * jnp.maximum(cy2 - cy1, 0.0)

            sv = jnp.zeros((16,), jnp.bool_)
            for t in range(7):
                x1 = work[pl.ds(KPAD + t * 16, 16)]
                y1 = work[pl.ds(2 * KPAD + t * 16, 16)]
                x2 = work[pl.ds(3 * KPAD + t * 16, 16)]
                y2 = work[pl.ds(4 * KPAD + t * 16, 16)]
                ar = work[pl.ds(5 * KPAD + t * 16, 16)]
                iw = jnp.maximum(jnp.minimum(cx2, x2) - jnp.maximum(cx1, x1), 0.0)
                ih = jnp.maximum(jnp.minimum(cy2, y2) - jnp.maximum(cy1, y1), 0.0)
                inter = ih * iw
                union = carea + ar - inter
                iou = inter / jnp.maximum(union, 1e-9)
                sv = sv | (iou > IOU_T)
            suppressed = jnp.any(sv)

            kmask = lane0 & valid
            plsc.store_scatter(sref, [idxv], _spl_f(NEG), mask=kmask)
            nm = jnp.max(jnp.where(lanes == lv, NEG, blk))
            plsc.store_scatter(bmax, [_spl_i(e)], _spl_f(nm), mask=kmask)

            take = valid & jnp.logical_not(suppressed)
            tmask = lane0 & take
            cntv = _spl_i(cnt)
            plsc.store_scatter(work, [cntv], _spl_f(gm), mask=tmask)
            plsc.store_scatter(work, [cntv + KPAD], cx1, mask=tmask)
            plsc.store_scatter(work, [cntv + 2 * KPAD], cy1, mask=tmask)
            plsc.store_scatter(work, [cntv + 3 * KPAD], cx2, mask=tmask)
            plsc.store_scatter(work, [cntv + 4 * KPAD], cy2, mask=tmask)
            plsc.store_scatter(work, [cntv + 5 * KPAD], carea, mask=tmask)

            return jnp.where(take, cnt + 1, cnt), jnp.logical_not(valid)

        lax.while_loop(cond, body, (jnp.int32(0), jnp.zeros((), jnp.bool_)))
        pltpu.sync_copy(work, shared.at[pl.ds(p * (6 * KPAD), 6 * KPAD)])

    p0, p1, p2 = sid, sid + 16, sid + 32
    start_load(p0, sref0, cref0, sem0)
    start_load(p1, sref1, cref1, sem1)
    wait_load(p0, sref0, cref0, sem0)
    run_problem(p0, sref0, cref0)

    @pl.when(sid < 8)
    def _():
        start_load(p2, sref0, cref0, sem0)

    wait_load(p1, sref1, cref1, sem1)
    run_problem(p1, sref1, cref1)

    @pl.when(sid < 8)
    def _():
        wait_load(p2, sref0, cref0, sem0)
        run_problem(p2, sref0, cref0)

    plsc.subcore_barrier()

    @pl.when(sid >= 14)
    def _():
        iml = sid - 14
        img = cid * 2 + iml
        pltpu.sync_copy(
            shared.at[pl.ds(iml * (NCLS * 6 * KPAD), NCLS * 6 * KPAD)], mrg)
        zeros16 = jnp.zeros((16,), jnp.int32)
        neg1 = jnp.full((16,), -1.0, jnp.float32)
        for t in range(KPAD // 16):
            os_v[pl.ds(t * 16, 16)] = neg1
            oc_v[pl.ds(t * 16, 16)] = neg1
        for t in range(KPAD * 4 // 16):
            ob_v[pl.ds(t * 16, 16)] = neg1
        cls_hi_ok = lanes < (NCLS - 16)
        cls1 = jnp.where(cls_hi_ok, lanes + 16, 0)

        def mcond(st):
            t, done, _, _ = st
            return (t < KMAX) & jnp.logical_not(done)

        def mbody(st):
            t, done, h0, h1 = st
            slot1 = jnp.where(cls_hi_ok, h1, KPAD - 1)
            v0 = plsc.load_gather(mrg, [lanes * (6 * KPAD) + h0])
            v1 = plsc.load_gather(mrg, [cls1 * (6 * KPAD) + slot1])
            gm = jnp.maximum(jnp.max(v0), jnp.max(v1))
            valid = gm > SCORE_T
            k0 = jnp.min(jnp.where(v0 == gm, lanes * KPAD + h0, BIG))
            k1 = jnp.min(jnp.where(v1 == gm, cls1 * KPAD + slot1, BIG))
            kbest = jnp.minimum(k0, k1)
            cbest = kbest // KPAD
            sbest = kbest % KPAD

            @pl.when(valid)
            def _():
                tv = _spl_i(t)
                plsc.store_scatter(os_v, [tv], _spl_f(gm), mask=lane0)
                plsc.store_scatter(oc_v, [tv],
                                   _spl_f(cbest.astype(jnp.float32)), mask=lane0)
                rows = 1 + (lanes & 3)
                vals = plsc.load_gather(
                    mrg, [cbest * (6 * KPAD) + rows * KPAD + sbest])
                plsc.store_scatter(ob_v, [t * 4 + (lanes & 3)], vals,
                                   mask=lanes < 4)

            upd = jnp.where(valid, jnp.int32(1), jnp.int32(0))
            h0n = h0 + jnp.where(lanes == cbest, upd, 0)
            h1n = h1 + jnp.where(lanes + 16 == cbest, upd, 0)
            return (jnp.where(valid, t + 1, t), jnp.logical_not(valid), h0n, h1n)

        tf, _, _, _ = lax.while_loop(
            mcond, mbody, (jnp.int32(0), jnp.zeros((), jnp.bool_), zeros16, zeros16))
        on_v[...] = jnp.where(lane0, tf, 0)
        pltpu.sync_copy(ob_v, outb_hbm.at[img])
        pltpu.sync_copy(os_v, outs_hbm.at[img])
        pltpu.sync_copy(oc_v, outc_hbm.at[img])
        pltpu.sync_copy(on_v, outn_hbm.at[img])


def kernel(box_prediction, confidence_prediction):
    conf_t = jnp.transpose(confidence_prediction, (0, 2, 1))
    conf_t = jnp.pad(conf_t, ((0, 0), (0, 0), (0, NPAD - NBOX)))
    coords = jnp.transpose(box_prediction, (0, 2, 1)).reshape(NIMG, 4 * NBOX)

    scores = pl.pallas_call(
        _softmax_body,
        grid=(NIMG,),
        in_specs=[pl.BlockSpec((1, NCLS, NPAD), lambda i: (i, 0, 0))],
        out_specs=pl.BlockSpec((1, NCLS, NPAD), lambda i: (i, 0, 0)),
        out_shape=jax.ShapeDtypeStruct((NIMG, NCLS, NPAD), jnp.float32),
    )(conf_t)

    mesh = plsc.VectorSubcoreMesh(core_axis_name="c", subcore_axis_name="s")
    nms = pl.kernel(
        _nms_body,
        mesh=mesh,
        compiler_params=pltpu.CompilerParams(needs_layout_passes=False),
        out_type=[
            jax.ShapeDtypeStruct((NIMG, KPAD * 4), jnp.float32),
            jax.ShapeDtypeStruct((NIMG, KPAD), jnp.float32),
            jax.ShapeDtypeStruct((NIMG, KPAD), jnp.float32),
            jax.ShapeDtypeStruct((NIMG, 16), jnp.int32),
        ],
        scratch_types=[
            pltpu.VMEM((NPAD,), jnp.float32),
            pltpu.VMEM((4 * NBOX,), jnp.float32),
            pltpu.VMEM((NPAD,), jnp.float32),
            pltpu.VMEM((4 * NBOX,), jnp.float32),
            pltpu.SemaphoreType.DMA,
            pltpu.SemaphoreType.DMA,
            pltpu.VMEM((NBLK,), jnp.float32),
            pltpu.VMEM((6 * KPAD,), jnp.float32),
            pltpu.VMEM((KPAD * 4,), jnp.float32),
            pltpu.VMEM((KPAD,), jnp.float32),
            pltpu.VMEM((KPAD,), jnp.float32),
            pltpu.VMEM((16,), jnp.int32),
            pltpu.VMEM((NCLS * 6 * KPAD,), jnp.float32),
            pltpu.VMEM_SHARED((40 * 6 * KPAD,), jnp.float32),
        ],
    )
    outb, outs, outc, outn = nms(scores, coords)
    out_b = outb.reshape(NIMG, KPAD, 4)[:, :KMAX, :]
    out_s = outs[:, :KMAX]
    out_c = outc[:, :KMAX]
    num_det = outn[:, 0]
    return out_b, out_s, out_c, num_det
